# Initial kernel scaffold; baseline (speedup 1.0000x reference)
import jax
import jax.numpy as jnp
from jax.experimental import pallas as pl


def kernel(x, edge_index, W1, b1, W2, b2):
    raise NotImplementedError("write your pallas kernel here")


# speedup vs baseline: 19.3991x; 19.3991x over previous
import functools

import jax
import jax.numpy as jnp
from jax import lax
from jax.experimental import pallas as pl
from jax.experimental.pallas import tpu as pltpu
from jax.experimental.pallas import tpu_sc as plsc

N = 10000
NPAD = 10240
E = 320000
D = 128
NW = 32
EPW = E // NW
CHUNK = 80
NCHUNK = EPW // CHUNK
IBLK = 25
NBLK = NCHUNK // IBLK

_mesh = plsc.VectorSubcoreMesh(core_axis_name="c", subcore_axis_name="s")


def _zero_fill_2d(buf, rows, cols):
    z16 = jnp.zeros((16,), jnp.float32)

    def row(i, _):
        def col(k, __):
            buf[i, pl.ds(k * 16, 16)] = z16
            return 0

        return lax.fori_loop(0, cols // 16, col, 0)

    lax.fori_loop(0, rows, row, 0)


@functools.partial(
    pl.kernel,
    out_type=jax.ShapeDtypeStruct((2 * NPAD,), jnp.float32),
    mesh=_mesh,
    scratch_types=[
        pltpu.VMEM((IBLK, CHUNK), jnp.int32),
        pltpu.VMEM((CHUNK,), jnp.float32),
        pltpu.VMEM((640,), jnp.float32),
        pltpu.VMEM_SHARED((NPAD,), jnp.float32),
    ],
)
def _sc_deg(dst_hbm, out_hbm, dst_v, ones_v, zbuf, deg_sp):
    c = lax.axis_index("c")
    s = lax.axis_index("s")
    w = c * 16 + s

    one16 = jnp.ones((16,), jnp.float32)
    z16 = jnp.zeros((16,), jnp.float32)

    def fill1(i, _):
        ones_v[pl.ds(i * 16, 16)] = one16
        return 0

    lax.fori_loop(0, CHUNK // 16, fill1, 0)

    def fill0(i, _):
        zbuf[pl.ds(i * 16, 16)] = z16
        return 0

    lax.fori_loop(0, 640 // 16, fill0, 0)

    pltpu.sync_copy(zbuf, deg_sp.at[pl.ds(s * 640, 640)])
    plsc.subcore_barrier()

    def blk(bi, _):
        pltpu.sync_copy(dst_hbm.at[w, bi], dst_v)

        def body(j, __):
            pltpu.sync_copy(ones_v, deg_sp.at[dst_v.at[j]], add=True)
            return 0

        return lax.fori_loop(0, IBLK, body, 0)

    lax.fori_loop(0, NBLK, blk, 0)
    plsc.subcore_barrier()

    pltpu.sync_copy(deg_sp.at[pl.ds(s * 640, 640)], zbuf)
    pltpu.sync_copy(zbuf, out_hbm.at[pl.ds(c * NPAD + s * 640, 640)])


@functools.partial(
    pl.kernel,
    out_type=jax.ShapeDtypeStruct((2 * NPAD, D), jnp.float32),
    mesh=_mesh,
    scratch_types=[
        pltpu.VMEM((IBLK, CHUNK), jnp.int32),
        pltpu.VMEM((IBLK, CHUNK), jnp.int32),
        pltpu.VMEM((CHUNK, D), jnp.float32),
        pltpu.VMEM((64, D), jnp.float32),
        pltpu.VMEM_SHARED((NPAD, D), jnp.float32),
        pltpu.SemaphoreType.DMA,
    ],
)
def _sc_scatter(g_hbm, src_hbm, dst_hbm, out_hbm, src_v, dst_v, rows_v,
                zbuf, acc_sp, sem):
    c = lax.axis_index("c")
    s = lax.axis_index("s")
    w = c * 16 + s

    _zero_fill_2d(zbuf, 64, D)
    for k in range(10):
        pltpu.sync_copy(zbuf, acc_sp.at[pl.ds(s * 640 + k * 64, 64)])
    plsc.subcore_barrier()

    def blk(bi, _):
        pltpu.sync_copy(src_hbm.at[w, bi], src_v)
        pltpu.sync_copy(dst_hbm.at[w, bi], dst_v)

        def body(j, __):
            pltpu.async_copy(g_hbm.at[src_v.at[j]], rows_v, sem).wait()
            pltpu.sync_copy(rows_v, acc_sp.at[dst_v.at[j]], add=True)
            return 0

        return lax.fori_loop(0, IBLK, body, 0)

    lax.fori_loop(0, NBLK, blk, 0)
    plsc.subcore_barrier()

    for k in range(10):
        r = s * 640 + k * 64
        pltpu.sync_copy(acc_sp.at[pl.ds(r, 64)], zbuf)
        pltpu.sync_copy(zbuf, out_hbm.at[pl.ds(c * NPAD + r, 64)])


BR = 1024
GRID = NPAD // BR


def _tc_lin_body(deg_ref, x_ref, w_ref, g_ref):
    dc = deg_ref[...]
    dis = lax.rsqrt(1.0 + dc[:, 0:1] + dc[:, 1:2])
    h = jnp.dot(x_ref[...], w_ref[...], preferred_element_type=jnp.float32)
    g_ref[...] = h * dis


def _tc_lin(degcol, xp, W):
    return pl.pallas_call(
        _tc_lin_body,
        grid=(GRID,),
        in_specs=[
            pl.BlockSpec((BR, 2), lambda i: (i, 0)),
            pl.BlockSpec((BR, D), lambda i: (i, 0)),
            pl.BlockSpec((D, D), lambda i: (0, 0)),
        ],
        out_specs=pl.BlockSpec((BR, D), lambda i: (i, 0)),
        out_shape=jax.ShapeDtypeStruct((NPAD, D), jnp.float32),
    )(degcol, xp, W)


def _tc_mid_body(s_ref, g1_ref, deg_ref, b_ref, w_ref, g2_ref):
    dc = deg_ref[...]
    dis = lax.rsqrt(1.0 + dc[:, 0:1] + dc[:, 1:2])
    ssum = s_ref[0] + s_ref[1] + g1_ref[...]
    y = jnp.maximum(dis * ssum + b_ref[...], 0.0)
    g2_ref[...] = jnp.dot(y, w_ref[...],
                          preferred_element_type=jnp.float32) * dis


def _tc_mid(s_parts, g1, degcol, b, W):
    return pl.pallas_call(
        _tc_mid_body,
        grid=(GRID,),
        in_specs=[
            pl.BlockSpec((2, BR, D), lambda i: (0, i, 0)),
            pl.BlockSpec((BR, D), lambda i: (i, 0)),
            pl.BlockSpec((BR, 2), lambda i: (i, 0)),
            pl.BlockSpec((1, D), lambda i: (0, 0)),
            pl.BlockSpec((D, D), lambda i: (0, 0)),
        ],
        out_specs=pl.BlockSpec((BR, D), lambda i: (i, 0)),
        out_shape=jax.ShapeDtypeStruct((NPAD, D), jnp.float32),
    )(s_parts, g1, degcol, b, W)


def _tc_out_body(s_ref, g2_ref, deg_ref, b_ref, o_ref):
    dc = deg_ref[...]
    dis = lax.rsqrt(1.0 + dc[:, 0:1] + dc[:, 1:2])
    ssum = s_ref[0] + s_ref[1] + g2_ref[...]
    o_ref[...] = dis * ssum + b_ref[...]


def _tc_out(s_parts, g2, degcol, b):
    return pl.pallas_call(
        _tc_out_body,
        grid=(GRID,),
        in_specs=[
            pl.BlockSpec((2, BR, D), lambda i: (0, i, 0)),
            pl.BlockSpec((BR, D), lambda i: (i, 0)),
            pl.BlockSpec((BR, 2), lambda i: (i, 0)),
            pl.BlockSpec((1, D), lambda i: (0, 0)),
        ],
        out_specs=pl.BlockSpec((BR, D), lambda i: (i, 0)),
        out_shape=jax.ShapeDtypeStruct((NPAD, D), jnp.float32),
    )(s_parts, g2, degcol, b)


def kernel(x, edge_index, W1, b1, W2, b2):
    src = edge_index[0].astype(jnp.int32).reshape(NW, NBLK, IBLK, CHUNK)
    dst = edge_index[1].astype(jnp.int32).reshape(NW, NBLK, IBLK, CHUNK)
    xp = jnp.pad(x, ((0, NPAD - N), (0, 0)))
    b1r = b1.reshape(1, D)
    b2r = b2.reshape(1, D)

    degcol = _sc_deg(dst).reshape(2, NPAD).T

    g1 = _tc_lin(degcol, xp, W1)
    s1 = _sc_scatter(g1, src, dst).reshape(2, NPAD, D)
    g2 = _tc_mid(s1, g1, degcol, b1r, W2)
    s2 = _sc_scatter(g2, src, dst).reshape(2, NPAD, D)
    out = _tc_out(s2, g2, degcol, b2r)
    return out[:N]


# speedup vs baseline: 32.3717x; 1.6687x over previous
import functools

import jax
import jax.numpy as jnp
from jax import lax
from jax.experimental import pallas as pl
from jax.experimental.pallas import tpu as pltpu
from jax.experimental.pallas import tpu_sc as plsc

N = 10000
NPAD = 10240
E = 320000
D = 128
NW = 32
EPW = E // NW
CHUNK = 100
NCHUNK = EPW // CHUNK
IBLK = 10
NBLK = NCHUNK // IBLK

_mesh = plsc.VectorSubcoreMesh(core_axis_name="c", subcore_axis_name="s")


def _zero_fill_2d(buf, rows, cols):
    z16 = jnp.zeros((16,), jnp.float32)

    def row(i, _):
        def col(k, __):
            buf[i, pl.ds(k * 16, 16)] = z16
            return 0

        return lax.fori_loop(0, cols // 16, col, 0)

    lax.fori_loop(0, rows, row, 0)


@functools.partial(
    pl.kernel,
    out_type=jax.ShapeDtypeStruct((2 * NPAD,), jnp.float32),
    mesh=_mesh,
    scratch_types=[
        pltpu.VMEM((IBLK, CHUNK), jnp.int32),
        pltpu.VMEM((CHUNK,), jnp.float32),
        pltpu.VMEM((640,), jnp.float32),
        pltpu.VMEM_SHARED((NPAD,), jnp.float32),
    ],
)
def _sc_deg(dst_hbm, out_hbm, dst_v, ones_v, zbuf, deg_sp):
    c = lax.axis_index("c")
    s = lax.axis_index("s")
    w = c * 16 + s

    one16 = jnp.ones((16,), jnp.float32)
    z16 = jnp.zeros((16,), jnp.float32)

    def fill1(i, _):
        ones_v[pl.ds(i * 16, 16)] = one16
        return 0

    lax.fori_loop(0, CHUNK // 16, fill1, 0)
    ones_v[pl.ds(CHUNK - 16, 16)] = one16

    def fill0(i, _):
        zbuf[pl.ds(i * 16, 16)] = z16
        return 0

    lax.fori_loop(0, 640 // 16, fill0, 0)

    pltpu.sync_copy(zbuf, deg_sp.at[pl.ds(s * 640, 640)])
    plsc.subcore_barrier()

    def blk(bi, _):
        pltpu.sync_copy(dst_hbm.at[w, bi], dst_v)

        def body(j, __):
            pltpu.sync_copy(ones_v, deg_sp.at[dst_v.at[j]], add=True)
            return 0

        return lax.fori_loop(0, IBLK, body, 0)

    lax.fori_loop(0, NBLK, blk, 0)
    plsc.subcore_barrier()

    pltpu.sync_copy(deg_sp.at[pl.ds(s * 640, 640)], zbuf)
    pltpu.sync_copy(zbuf, out_hbm.at[pl.ds(c * NPAD + s * 640, 640)])


@functools.partial(
    pl.kernel,
    out_type=jax.ShapeDtypeStruct((2 * NPAD, D), jnp.float32),
    mesh=_mesh,
    scratch_types=[
        pltpu.VMEM((2 * IBLK, CHUNK), jnp.int32),
        pltpu.VMEM((2 * IBLK, CHUNK), jnp.int32),
        pltpu.VMEM((CHUNK, D), jnp.float32),
        pltpu.VMEM((CHUNK, D), jnp.float32),
        pltpu.VMEM((CHUNK, D), jnp.float32),
        pltpu.VMEM((16, D), jnp.float32),
        pltpu.VMEM_SHARED((NPAD, D), jnp.float32),
        pltpu.SemaphoreType.DMA,
        pltpu.SemaphoreType.DMA,
        pltpu.SemaphoreType.DMA,
        pltpu.SemaphoreType.DMA,
        pltpu.SemaphoreType.DMA,
        pltpu.SemaphoreType.DMA,
    ],
)
def _sc_scatter(g_hbm, src_hbm, dst_hbm, out_hbm, src_v, dst_v, rows0,
                rows1, rows2, zbuf, acc_sp, gsem0, gsem1, gsem2, ssem0,
                ssem1, ssem2):
    c = lax.axis_index("c")
    s = lax.axis_index("s")
    w = c * 16 + s
    bufs = (rows0, rows1, rows2)
    gsems = (gsem0, gsem1, gsem2)
    ssems = (ssem0, ssem1, ssem2)

    pltpu.sync_copy(src_hbm.at[w, 0], src_v.at[pl.ds(0, IBLK)])
    pltpu.async_copy(g_hbm.at[src_v.at[0]], rows0, gsem0)
    pltpu.async_copy(g_hbm.at[src_v.at[1]], rows1, gsem1)

    _zero_fill_2d(zbuf, 16, D)
    for k in range(40):
        pltpu.sync_copy(zbuf, acc_sp.at[pl.ds(s * 640 + k * 16, 16)])
    plsc.subcore_barrier()

    def gwait(p):
        pltpu.make_async_copy(g_hbm.at[src_v.at[0]], bufs[p],
                              gsems[p]).wait()

    def swait(p):
        pltpu.make_async_copy(bufs[p], acc_sp.at[dst_v.at[0]],
                              ssems[p]).wait()

    def step(j, p):
        @pl.when(j % IBLK == 0)
        def _():
            half = (j // IBLK) % 2
            pltpu.sync_copy(dst_hbm.at[w, j // IBLK],
                            dst_v.at[pl.ds(half * IBLK, IBLK)])

        gwait(p)
        pltpu.async_copy(bufs[p], acc_sp.at[dst_v.at[j % (2 * IBLK)]],
                         ssems[p], add=True)
        nj = j + 2

        @pl.when(nj < NCHUNK)
        def _():
            @pl.when(nj % IBLK == 0)
            def __():
                half = (nj // IBLK) % 2
                pltpu.sync_copy(src_hbm.at[w, nj // IBLK],
                                src_v.at[pl.ds(half * IBLK, IBLK)])

            @pl.when(j >= 1)
            def ___():
                swait((p + 2) % 3)

            q = (p + 2) % 3
            pltpu.async_copy(g_hbm.at[src_v.at[nj % (2 * IBLK)]],
                             bufs[q], gsems[q])

    def triple(jj, _):
        step(3 * jj, 0)

        @pl.when(3 * jj + 1 < NCHUNK)
        def _():
            step(3 * jj + 1, 1)

        @pl.when(3 * jj + 2 < NCHUNK)
        def _():
            step(3 * jj + 2, 2)

        return 0

    lax.fori_loop(0, (NCHUNK + 2) // 3, triple, 0)
    swait((NCHUNK - 3) % 3)
    swait((NCHUNK - 2) % 3)
    swait((NCHUNK - 1) % 3)
    plsc.subcore_barrier()

    pltpu.sync_copy(acc_sp.at[pl.ds(s * 640, 640)],
                    out_hbm.at[pl.ds(c * NPAD + s * 640, 640)])


BR = 1024
GRID = NPAD // BR


def _tc_lin_body(deg_ref, x_ref, w_ref, g_ref):
    dc = deg_ref[...]
    dis = lax.rsqrt(1.0 + dc[:, 0:1] + dc[:, 1:2])
    h = jnp.dot(x_ref[...], w_ref[...], preferred_element_type=jnp.float32)
    g_ref[...] = h * dis


def _tc_lin(degcol, xp, W):
    return pl.pallas_call(
        _tc_lin_body,
        grid=(GRID,),
        in_specs=[
            pl.BlockSpec((BR, 2), lambda i: (i, 0)),
            pl.BlockSpec((BR, D), lambda i: (i, 0)),
            pl.BlockSpec((D, D), lambda i: (0, 0)),
        ],
        out_specs=pl.BlockSpec((BR, D), lambda i: (i, 0)),
        out_shape=jax.ShapeDtypeStruct((NPAD, D), jnp.float32),
    )(degcol, xp, W)


def _tc_mid_body(s_ref, g1_ref, deg_ref, b_ref, w_ref, g2_ref):
    dc = deg_ref[...]
    dis = lax.rsqrt(1.0 + dc[:, 0:1] + dc[:, 1:2])
    ssum = s_ref[0] + s_ref[1] + g1_ref[...]
    y = jnp.maximum(dis * ssum + b_ref[...], 0.0)
    g2_ref[...] = jnp.dot(y, w_ref[...],
                          preferred_element_type=jnp.float32) * dis


def _tc_mid(s_parts, g1, degcol, b, W):
    return pl.pallas_call(
        _tc_mid_body,
        grid=(GRID,),
        in_specs=[
            pl.BlockSpec((2, BR, D), lambda i: (0, i, 0)),
            pl.BlockSpec((BR, D), lambda i: (i, 0)),
            pl.BlockSpec((BR, 2), lambda i: (i, 0)),
            pl.BlockSpec((1, D), lambda i: (0, 0)),
            pl.BlockSpec((D, D), lambda i: (0, 0)),
        ],
        out_specs=pl.BlockSpec((BR, D), lambda i: (i, 0)),
        out_shape=jax.ShapeDtypeStruct((NPAD, D), jnp.float32),
    )(s_parts, g1, degcol, b, W)


def _tc_out_body(s_ref, g2_ref, deg_ref, b_ref, o_ref):
    dc = deg_ref[...]
    dis = lax.rsqrt(1.0 + dc[:, 0:1] + dc[:, 1:2])
    ssum = s_ref[0] + s_ref[1] + g2_ref[...]
    o_ref[...] = dis * ssum + b_ref[...]


def _tc_out(s_parts, g2, degcol, b):
    return pl.pallas_call(
        _tc_out_body,
        grid=(GRID,),
        in_specs=[
            pl.BlockSpec((2, BR, D), lambda i: (0, i, 0)),
            pl.BlockSpec((BR, D), lambda i: (i, 0)),
            pl.BlockSpec((BR, 2), lambda i: (i, 0)),
            pl.BlockSpec((1, D), lambda i: (0, 0)),
        ],
        out_specs=pl.BlockSpec((BR, D), lambda i: (i, 0)),
        out_shape=jax.ShapeDtypeStruct((NPAD, D), jnp.float32),
    )(s_parts, g2, degcol, b)


def kernel(x, edge_index, W1, b1, W2, b2):
    src = edge_index[0].astype(jnp.int32).reshape(NW, NBLK, IBLK, CHUNK)
    dst = edge_index[1].astype(jnp.int32).reshape(NW, NBLK, IBLK, CHUNK)
    xp = jnp.pad(x, ((0, NPAD - N), (0, 0)))
    b1r = b1.reshape(1, D)
    b2r = b2.reshape(1, D)

    degcol = _sc_deg(dst).reshape(2, NPAD).T

    g1 = _tc_lin(degcol, xp, W1)
    s1 = _sc_scatter(g1, src, dst).reshape(2, NPAD, D)
    g2 = _tc_mid(s1, g1, degcol, b1r, W2)
    s2 = _sc_scatter(g2, src, dst).reshape(2, NPAD, D)
    out = _tc_out(s2, g2, degcol, b2r)
    return out[:N]


# speedup vs baseline: 34.1085x; 1.0537x over previous
import functools

import jax
import jax.numpy as jnp
from jax import lax
from jax.experimental import pallas as pl
from jax.experimental.pallas import tpu as pltpu
from jax.experimental.pallas import tpu_sc as plsc

N = 10000
NPAD = 10240
E = 320000
D = 128
NW = 32
EPW = E // NW
CHUNK = 80
NCHUNK = EPW // CHUNK
IBLK = 25
NBLK = NCHUNK // IBLK

_mesh = plsc.VectorSubcoreMesh(core_axis_name="c", subcore_axis_name="s")


def _zero_fill_2d(buf, rows, cols):
    z16 = jnp.zeros((16,), jnp.float32)

    def row(i, _):
        def col(k, __):
            buf[i, pl.ds(k * 16, 16)] = z16
            return 0

        return lax.fori_loop(0, cols // 16, col, 0)

    lax.fori_loop(0, rows, row, 0)


@functools.partial(
    pl.kernel,
    out_type=jax.ShapeDtypeStruct((2 * NPAD,), jnp.float32),
    mesh=_mesh,
    scratch_types=[
        pltpu.VMEM((2 * IBLK, CHUNK), jnp.int32),
        pltpu.VMEM((CHUNK,), jnp.float32),
        pltpu.VMEM((640,), jnp.float32),
        pltpu.VMEM_SHARED((NPAD,), jnp.float32),
        pltpu.SemaphoreType.DMA,
        pltpu.SemaphoreType.DMA,
    ],
)
def _sc_deg(dst_hbm, out_hbm, dst_v, ones_v, zbuf, deg_sp, dsem0, dsem1):
    c = lax.axis_index("c")
    s = lax.axis_index("s")
    w = c * 16 + s
    dsems = (dsem0, dsem1)

    one16 = jnp.ones((16,), jnp.float32)
    z16 = jnp.zeros((16,), jnp.float32)

    def fill1(i, _):
        ones_v[pl.ds(i * 16, 16)] = one16
        return 0

    lax.fori_loop(0, CHUNK // 16, fill1, 0)

    def fill0(i, _):
        zbuf[pl.ds(i * 16, 16)] = z16
        return 0

    lax.fori_loop(0, 640 // 16, fill0, 0)

    pltpu.sync_copy(zbuf, deg_sp.at[pl.ds(s * 640, 640)])
    plsc.subcore_barrier()

    def drain(h):
        def one(j, _):
            pltpu.make_async_copy(
                ones_v, deg_sp.at[dst_v.at[h * IBLK]], dsems[h]).wait()
            return 0

        return lax.fori_loop(0, IBLK, one, 0)

    for bi in range(NBLK):
        h = bi % 2
        if bi >= 2:
            drain(h)
        pltpu.sync_copy(dst_hbm.at[w, bi],
                        dst_v.at[pl.ds(h * IBLK, IBLK)])

        def fire(j, _, bi=bi, h=h):
            pltpu.async_copy(ones_v, deg_sp.at[dst_v.at[h * IBLK + j]],
                             dsems[h], add=True)
            return 0

        lax.fori_loop(0, IBLK, fire, 0)

    drain((NBLK - 2) % 2)
    drain((NBLK - 1) % 2)
    plsc.subcore_barrier()

    pltpu.sync_copy(deg_sp.at[pl.ds(s * 640, 640)], zbuf)
    pltpu.sync_copy(zbuf, out_hbm.at[pl.ds(c * NPAD + s * 640, 640)])


@functools.partial(
    pl.kernel,
    out_type=jax.ShapeDtypeStruct((2 * NPAD, D), jnp.float32),
    mesh=_mesh,
    scratch_types=[
        pltpu.VMEM((2 * IBLK, CHUNK), jnp.int32),
        pltpu.VMEM((2 * IBLK, CHUNK), jnp.int32),
        pltpu.VMEM((CHUNK, D), jnp.float32),
        pltpu.VMEM((CHUNK, D), jnp.float32),
        pltpu.VMEM((CHUNK, D), jnp.float32),
        pltpu.VMEM((32, D), jnp.float32),
        pltpu.VMEM_SHARED((NPAD, D), jnp.float32),
        pltpu.SemaphoreType.DMA,
        pltpu.SemaphoreType.DMA,
        pltpu.SemaphoreType.DMA,
        pltpu.SemaphoreType.DMA,
        pltpu.SemaphoreType.DMA,
        pltpu.SemaphoreType.DMA,
    ],
)
def _sc_scatter(g_hbm, src_hbm, dst_hbm, out_hbm, src_v, dst_v, rows0,
                rows1, rows2, zbuf, acc_sp, gsem0, gsem1, gsem2, ssem0,
                ssem1, ssem2):
    c = lax.axis_index("c")
    s = lax.axis_index("s")
    w = c * 16 + s
    bufs = (rows0, rows1, rows2)
    gsems = (gsem0, gsem1, gsem2)
    ssems = (ssem0, ssem1, ssem2)

    _zero_fill_2d(zbuf, 32, D)
    for k in range(20):
        pltpu.sync_copy(zbuf, acc_sp.at[pl.ds(s * 640 + k * 32, 32)])
    plsc.subcore_barrier()

    def gwait(p):
        pltpu.make_async_copy(g_hbm.at[pl.ds(0, CHUNK)], bufs[p],
                              gsems[p]).wait()

    def swait(p):
        pltpu.make_async_copy(bufs[p], acc_sp.at[pl.ds(0, CHUNK)],
                              ssems[p]).wait()

    pltpu.sync_copy(src_hbm.at[w, 0], src_v.at[pl.ds(0, IBLK)])
    pltpu.async_copy(g_hbm.at[src_v.at[0]], rows0, gsem0)
    pltpu.async_copy(g_hbm.at[src_v.at[1]], rows1, gsem1)

    def step(j, p):
        @pl.when(j % IBLK == 0)
        def _():
            half = (j // IBLK) % 2
            pltpu.sync_copy(dst_hbm.at[w, j // IBLK],
                            dst_v.at[pl.ds(half * IBLK, IBLK)])

        gwait(p)
        pltpu.async_copy(bufs[p], acc_sp.at[dst_v.at[j % (2 * IBLK)]],
                         ssems[p], add=True)
        nj = j + 2

        @pl.when(nj < NCHUNK)
        def _():
            @pl.when(nj % IBLK == 0)
            def __():
                half = (nj // IBLK) % 2
                pltpu.sync_copy(src_hbm.at[w, nj // IBLK],
                                src_v.at[pl.ds(half * IBLK, IBLK)])

            @pl.when(j >= 1)
            def ___():
                swait((p + 2) % 3)

            q = (p + 2) % 3
            pltpu.async_copy(g_hbm.at[src_v.at[nj % (2 * IBLK)]],
                             bufs[q], gsems[q])

    def triple(jj, _):
        step(3 * jj, 0)

        @pl.when(3 * jj + 1 < NCHUNK)
        def _():
            step(3 * jj + 1, 1)

        @pl.when(3 * jj + 2 < NCHUNK)
        def _():
            step(3 * jj + 2, 2)

        return 0

    lax.fori_loop(0, (NCHUNK + 2) // 3, triple, 0)
    swait((NCHUNK - 3) % 3)
    swait((NCHUNK - 2) % 3)
    swait((NCHUNK - 1) % 3)
    plsc.subcore_barrier()

    for k in range(20):
        r = s * 640 + k * 32
        pltpu.sync_copy(acc_sp.at[pl.ds(r, 32)], zbuf)
        pltpu.sync_copy(zbuf, out_hbm.at[pl.ds(c * NPAD + r, 32)])


BR = 1024
GRID = NPAD // BR


def _tc_lin_body(deg_ref, x_ref, w_ref, g_ref):
    dc = deg_ref[...]
    dis = lax.rsqrt(1.0 + dc[:, 0:1] + dc[:, 1:2])
    h = jnp.dot(x_ref[...], w_ref[...], preferred_element_type=jnp.float32)
    g_ref[...] = h * dis


def _tc_lin(degcol, xp, W):
    return pl.pallas_call(
        _tc_lin_body,
        grid=(GRID,),
        in_specs=[
            pl.BlockSpec((BR, 2), lambda i: (i, 0)),
            pl.BlockSpec((BR, D), lambda i: (i, 0)),
            pl.BlockSpec((D, D), lambda i: (0, 0)),
        ],
        out_specs=pl.BlockSpec((BR, D), lambda i: (i, 0)),
        out_shape=jax.ShapeDtypeStruct((NPAD, D), jnp.float32),
    )(degcol, xp, W)


def _tc_mid_body(s_ref, g1_ref, deg_ref, b_ref, w_ref, g2_ref):
    dc = deg_ref[...]
    dis = lax.rsqrt(1.0 + dc[:, 0:1] + dc[:, 1:2])
    ssum = s_ref[0] + s_ref[1] + g1_ref[...]
    y = jnp.maximum(dis * ssum + b_ref[...], 0.0)
    g2_ref[...] = jnp.dot(y, w_ref[...],
                          preferred_element_type=jnp.float32) * dis


def _tc_mid(s_parts, g1, degcol, b, W):
    return pl.pallas_call(
        _tc_mid_body,
        grid=(GRID,),
        in_specs=[
            pl.BlockSpec((2, BR, D), lambda i: (0, i, 0)),
            pl.BlockSpec((BR, D), lambda i: (i, 0)),
            pl.BlockSpec((BR, 2), lambda i: (i, 0)),
            pl.BlockSpec((1, D), lambda i: (0, 0)),
            pl.BlockSpec((D, D), lambda i: (0, 0)),
        ],
        out_specs=pl.BlockSpec((BR, D), lambda i: (i, 0)),
        out_shape=jax.ShapeDtypeStruct((NPAD, D), jnp.float32),
    )(s_parts, g1, degcol, b, W)


def _tc_out_body(s_ref, g2_ref, deg_ref, b_ref, o_ref):
    dc = deg_ref[...]
    dis = lax.rsqrt(1.0 + dc[:, 0:1] + dc[:, 1:2])
    ssum = s_ref[0] + s_ref[1] + g2_ref[...]
    o_ref[...] = dis * ssum + b_ref[...]


def _tc_out(s_parts, g2, degcol, b):
    return pl.pallas_call(
        _tc_out_body,
        grid=(GRID,),
        in_specs=[
            pl.BlockSpec((2, BR, D), lambda i: (0, i, 0)),
            pl.BlockSpec((BR, D), lambda i: (i, 0)),
            pl.BlockSpec((BR, 2), lambda i: (i, 0)),
            pl.BlockSpec((1, D), lambda i: (0, 0)),
        ],
        out_specs=pl.BlockSpec((BR, D), lambda i: (i, 0)),
        out_shape=jax.ShapeDtypeStruct((N, D), jnp.float32),
    )(s_parts, g2, degcol, b)


def kernel(x, edge_index, W1, b1, W2, b2):
    src = edge_index[0].astype(jnp.int32).reshape(NW, NBLK, IBLK, CHUNK)
    dst = edge_index[1].astype(jnp.int32).reshape(NW, NBLK, IBLK, CHUNK)
    b1r = b1.reshape(1, D)
    b2r = b2.reshape(1, D)

    degcol = _sc_deg(dst).reshape(2, NPAD).T

    g1 = _tc_lin(degcol, x, W1)
    s1 = _sc_scatter(g1, src, dst).reshape(2, NPAD, D)
    g2 = _tc_mid(s1, g1, degcol, b1r, W2)
    s2 = _sc_scatter(g2, src, dst).reshape(2, NPAD, D)
    return _tc_out(s2, g2, degcol, b2r)


# speedup vs baseline: 35.0209x; 1.0267x over previous
import functools

import jax
import jax.numpy as jnp
from jax import lax
from jax.experimental import pallas as pl
from jax.experimental.pallas import tpu as pltpu
from jax.experimental.pallas import tpu_sc as plsc

N = 10000
NPAD = 10240
E = 320000
D = 128
NW = 32
EPW = E // NW
CHUNK = 80
NCHUNK = EPW // CHUNK
IBLK = 25
NBLK = NCHUNK // IBLK

_mesh = plsc.VectorSubcoreMesh(core_axis_name="c", subcore_axis_name="s")


def _zero_fill_2d(buf, rows, cols):
    z16 = jnp.zeros((16,), jnp.float32)

    def row(i, _):
        def col(k, __):
            buf[i, pl.ds(k * 16, 16)] = z16
            return 0

        return lax.fori_loop(0, cols // 16, col, 0)

    lax.fori_loop(0, rows, row, 0)


@functools.partial(
    pl.kernel,
    out_type=jax.ShapeDtypeStruct((2 * NPAD,), jnp.float32),
    mesh=_mesh,
    scratch_types=[
        pltpu.VMEM((2 * IBLK, CHUNK), jnp.int32),
        pltpu.VMEM((CHUNK,), jnp.float32),
        pltpu.VMEM((640,), jnp.float32),
        pltpu.VMEM_SHARED((NPAD,), jnp.float32),
        pltpu.SemaphoreType.DMA,
        pltpu.SemaphoreType.DMA,
    ],
)
def _sc_deg(dst_hbm, out_hbm, dst_v, ones_v, zbuf, deg_sp, dsem0, dsem1):
    c = lax.axis_index("c")
    s = lax.axis_index("s")
    w = c * 16 + s
    dsems = (dsem0, dsem1)

    one16 = jnp.ones((16,), jnp.float32)
    z16 = jnp.zeros((16,), jnp.float32)

    def fill1(i, _):
        ones_v[pl.ds(i * 16, 16)] = one16
        return 0

    lax.fori_loop(0, CHUNK // 16, fill1, 0)

    def fill0(i, _):
        zbuf[pl.ds(i * 16, 16)] = z16
        return 0

    lax.fori_loop(0, 640 // 16, fill0, 0)

    pltpu.sync_copy(zbuf, deg_sp.at[pl.ds(s * 640, 640)])
    plsc.subcore_barrier()

    def drain(h):
        def one(j, _):
            pltpu.make_async_copy(
                ones_v, deg_sp.at[dst_v.at[h * IBLK]], dsems[h]).wait()
            return 0

        return lax.fori_loop(0, IBLK, one, 0)

    for bi in range(NBLK):
        h = bi % 2
        if bi >= 2:
            drain(h)
        pltpu.sync_copy(dst_hbm.at[w, bi],
                        dst_v.at[pl.ds(h * IBLK, IBLK)])

        def fire(j, _, bi=bi, h=h):
            pltpu.async_copy(ones_v, deg_sp.at[dst_v.at[h * IBLK + j]],
                             dsems[h], add=True)
            return 0

        lax.fori_loop(0, IBLK, fire, 0)

    drain((NBLK - 2) % 2)
    drain((NBLK - 1) % 2)
    plsc.subcore_barrier()

    pltpu.sync_copy(deg_sp.at[pl.ds(s * 640, 640)], zbuf)
    pltpu.sync_copy(zbuf, out_hbm.at[pl.ds(c * NPAD + s * 640, 640)])


@functools.partial(
    pl.kernel,
    out_type=jax.ShapeDtypeStruct((2 * NPAD, D), jnp.float32),
    mesh=_mesh,
    scratch_types=[
        pltpu.VMEM((2 * IBLK, CHUNK), jnp.int32),
        pltpu.VMEM((2 * IBLK, CHUNK), jnp.int32),
        pltpu.VMEM((CHUNK, D), jnp.float32),
        pltpu.VMEM((CHUNK, D), jnp.float32),
        pltpu.VMEM((CHUNK, D), jnp.float32),
        pltpu.VMEM((32, D), jnp.float32),
        pltpu.VMEM_SHARED((NPAD, D), jnp.float32),
        pltpu.SemaphoreType.DMA,
        pltpu.SemaphoreType.DMA,
        pltpu.SemaphoreType.DMA,
        pltpu.SemaphoreType.DMA,
        pltpu.SemaphoreType.DMA,
        pltpu.SemaphoreType.DMA,
    ],
)
def _sc_scatter(g_hbm, src_hbm, dst_hbm, out_hbm, src_v, dst_v, rows0,
                rows1, rows2, zbuf, acc_sp, gsem0, gsem1, gsem2, ssem0,
                ssem1, ssem2):
    c = lax.axis_index("c")
    s = lax.axis_index("s")
    w = c * 16 + s
    bufs = (rows0, rows1, rows2)
    gsems = (gsem0, gsem1, gsem2)
    ssems = (ssem0, ssem1, ssem2)

    pltpu.sync_copy(src_hbm.at[w, 0], src_v.at[pl.ds(0, IBLK)])
    pltpu.async_copy(g_hbm.at[src_v.at[0]], rows0, gsem0)
    pltpu.async_copy(g_hbm.at[src_v.at[1]], rows1, gsem1)

    _zero_fill_2d(zbuf, 32, D)
    for k in range(20):
        pltpu.sync_copy(zbuf, acc_sp.at[pl.ds(s * 640 + k * 32, 32)])
    plsc.subcore_barrier()

    def gwait(p):
        pltpu.make_async_copy(g_hbm.at[pl.ds(0, CHUNK)], bufs[p],
                              gsems[p]).wait()

    def swait(p):
        pltpu.make_async_copy(bufs[p], acc_sp.at[pl.ds(0, CHUNK)],
                              ssems[p]).wait()

    def step(j, p):
        @pl.when(j % IBLK == 0)
        def _():
            half = (j // IBLK) % 2
            pltpu.sync_copy(dst_hbm.at[w, j // IBLK],
                            dst_v.at[pl.ds(half * IBLK, IBLK)])

        gwait(p)
        pltpu.async_copy(bufs[p], acc_sp.at[dst_v.at[j % (2 * IBLK)]],
                         ssems[p], add=True)
        nj = j + 2

        @pl.when(nj < NCHUNK)
        def _():
            @pl.when(nj % IBLK == 0)
            def __():
                half = (nj // IBLK) % 2
                pltpu.sync_copy(src_hbm.at[w, nj // IBLK],
                                src_v.at[pl.ds(half * IBLK, IBLK)])

            @pl.when(j >= 1)
            def ___():
                swait((p + 2) % 3)

            q = (p + 2) % 3
            pltpu.async_copy(g_hbm.at[src_v.at[nj % (2 * IBLK)]],
                             bufs[q], gsems[q])

    def triple(jj, _):
        step(3 * jj, 0)

        @pl.when(3 * jj + 1 < NCHUNK)
        def _():
            step(3 * jj + 1, 1)

        @pl.when(3 * jj + 2 < NCHUNK)
        def _():
            step(3 * jj + 2, 2)

        return 0

    lax.fori_loop(0, (NCHUNK + 2) // 3, triple, 0)
    swait((NCHUNK - 3) % 3)
    swait((NCHUNK - 2) % 3)
    swait((NCHUNK - 1) % 3)
    plsc.subcore_barrier()

    pltpu.sync_copy(acc_sp.at[pl.ds(s * 640, 640)],
                    out_hbm.at[pl.ds(c * NPAD + s * 640, 640)])


BR = 1024
GRID = NPAD // BR


def _tc_lin_body(deg_ref, x_ref, w_ref, g_ref):
    dc = deg_ref[...]
    dis = lax.rsqrt(1.0 + dc[:, 0:1] + dc[:, 1:2])
    h = jnp.dot(x_ref[...], w_ref[...], preferred_element_type=jnp.float32)
    g_ref[...] = h * dis


def _tc_lin(degcol, xp, W):
    return pl.pallas_call(
        _tc_lin_body,
        grid=(GRID,),
        in_specs=[
            pl.BlockSpec((BR, 2), lambda i: (i, 0)),
            pl.BlockSpec((BR, D), lambda i: (i, 0)),
            pl.BlockSpec((D, D), lambda i: (0, 0)),
        ],
        out_specs=pl.BlockSpec((BR, D), lambda i: (i, 0)),
        out_shape=jax.ShapeDtypeStruct((NPAD, D), jnp.float32),
    )(degcol, xp, W)


def _tc_mid_body(s_ref, g1_ref, deg_ref, b_ref, w_ref, g2_ref):
    dc = deg_ref[...]
    dis = lax.rsqrt(1.0 + dc[:, 0:1] + dc[:, 1:2])
    ssum = s_ref[0] + s_ref[1] + g1_ref[...]
    y = jnp.maximum(dis * ssum + b_ref[...], 0.0)
    g2_ref[...] = jnp.dot(y, w_ref[...],
                          preferred_element_type=jnp.float32) * dis


def _tc_mid(s_parts, g1, degcol, b, W):
    return pl.pallas_call(
        _tc_mid_body,
        grid=(GRID,),
        in_specs=[
            pl.BlockSpec((2, BR, D), lambda i: (0, i, 0)),
            pl.BlockSpec((BR, D), lambda i: (i, 0)),
            pl.BlockSpec((BR, 2), lambda i: (i, 0)),
            pl.BlockSpec((1, D), lambda i: (0, 0)),
            pl.BlockSpec((D, D), lambda i: (0, 0)),
        ],
        out_specs=pl.BlockSpec((BR, D), lambda i: (i, 0)),
        out_shape=jax.ShapeDtypeStruct((NPAD, D), jnp.float32),
    )(s_parts, g1, degcol, b, W)


def _tc_out_body(s_ref, g2_ref, deg_ref, b_ref, o_ref):
    dc = deg_ref[...]
    dis = lax.rsqrt(1.0 + dc[:, 0:1] + dc[:, 1:2])
    ssum = s_ref[0] + s_ref[1] + g2_ref[...]
    o_ref[...] = dis * ssum + b_ref[...]


def _tc_out(s_parts, g2, degcol, b):
    return pl.pallas_call(
        _tc_out_body,
        grid=(GRID,),
        in_specs=[
            pl.BlockSpec((2, BR, D), lambda i: (0, i, 0)),
            pl.BlockSpec((BR, D), lambda i: (i, 0)),
            pl.BlockSpec((BR, 2), lambda i: (i, 0)),
            pl.BlockSpec((1, D), lambda i: (0, 0)),
        ],
        out_specs=pl.BlockSpec((BR, D), lambda i: (i, 0)),
        out_shape=jax.ShapeDtypeStruct((N, D), jnp.float32),
    )(s_parts, g2, degcol, b)


def kernel(x, edge_index, W1, b1, W2, b2):
    src = edge_index[0].astype(jnp.int32).reshape(NW, NBLK, IBLK, CHUNK)
    dst = edge_index[1].astype(jnp.int32).reshape(NW, NBLK, IBLK, CHUNK)
    b1r = b1.reshape(1, D)
    b2r = b2.reshape(1, D)

    degcol = _sc_deg(dst).reshape(2, NPAD).T

    g1 = _tc_lin(degcol, x, W1)
    s1 = _sc_scatter(g1, src, dst).reshape(2, NPAD, D)
    g2 = _tc_mid(s1, g1, degcol, b1r, W2)
    s2 = _sc_scatter(g2, src, dst).reshape(2, NPAD, D)
    return _tc_out(s2, g2, degcol, b2r)
